# initial kernel scaffold (unmeasured)
import functools
import math

import jax
import jax.numpy as jnp
from jax import lax
from jax.experimental import pallas as pl
from jax.experimental.pallas import tpu as pltpu

N_DEV = 32


def kernel(q, k, v):
    s_per, d = q.shape
    scale = 1.0 / math.sqrt(d)

    def body(q_ref, k_ref, v_ref, out_ref, kv_ref, send_sems, recv_sems,
             credit_sem):
        my = lax.axis_index("i")
        left = (my - 1) % N_DEV
        right = (my + 1) % N_DEV

        barrier_sem = pltpu.get_barrier_semaphore()
        for nbr in [left, right]:
            pl.semaphore_signal(
                barrier_sem, inc=1,
                device_id=(nbr,), device_id_type=pl.DeviceIdType.MESH,
            )
        pl.semaphore_wait(barrier_sem, 2)

        kv_ref[0, pl.ds(0, s_per), :] = k_ref[:, :].astype(jnp.bfloat16)
        kv_ref[0, pl.ds(s_per, s_per), :] = v_ref[:, :].astype(jnp.bfloat16)

        q_bf = (q_ref[:, :] * scale).astype(jnp.bfloat16)

        def block_update(state, k_blk, v_blk):
            m, l, acc = state
            s = lax.dot_general(
                q_bf, k_blk, (((1,), (1,)), ((), ())),
                preferred_element_type=jnp.float32,
            )
            m_new = jnp.maximum(m, jnp.max(s, axis=1, keepdims=True))
            p = jnp.exp(s - m_new)
            alpha = jnp.exp(m - m_new)
            l_new = l * alpha + jnp.sum(p, axis=1, keepdims=True)
            acc_new = acc * alpha + lax.dot_general(
                p.astype(jnp.bfloat16), v_blk, (((1,), (0,)), ((), ())),
                preferred_element_type=jnp.float32,
            )
            return m_new, l_new, acc_new

        m0 = jnp.full((s_per, 1), -jnp.inf, dtype=jnp.float32)
        l0 = jnp.zeros((s_per, 1), dtype=jnp.float32)
        a0 = jnp.zeros((s_per, d), dtype=jnp.float32)
        state = block_update(
            (m0, l0, a0),
            k_ref[:, :].astype(jnp.bfloat16),
            v_ref[:, :].astype(jnp.bfloat16),
        )

        for h in range(N_DEV - 1):
            send_slot = h % 2
            recv_slot = (h + 1) % 2
            if h >= 1:
                pl.semaphore_wait(credit_sem, 1)
            rdma = pltpu.make_async_remote_copy(
                src_ref=kv_ref.at[send_slot],
                dst_ref=kv_ref.at[recv_slot],
                send_sem=send_sems.at[send_slot],
                recv_sem=recv_sems.at[recv_slot],
                device_id=(right,),
                device_id_type=pl.DeviceIdType.MESH,
            )
            rdma.start()
            rdma.wait()
            pl.semaphore_signal(
                credit_sem, inc=1,
                device_id=(left,), device_id_type=pl.DeviceIdType.MESH,
            )
            state = block_update(
                state,
                kv_ref[recv_slot, pl.ds(0, s_per), :],
                kv_ref[recv_slot, pl.ds(s_per, s_per), :],
            )

        m, l, acc = state
        out_ref[:, :] = acc / l

    return pl.pallas_call(
        body,
        out_shape=jax.ShapeDtypeStruct((s_per, d), jnp.float32),
        in_specs=[
            pl.BlockSpec(memory_space=pltpu.VMEM),
            pl.BlockSpec(memory_space=pltpu.VMEM),
            pl.BlockSpec(memory_space=pltpu.VMEM),
        ],
        out_specs=pl.BlockSpec(memory_space=pltpu.VMEM),
        scratch_shapes=[
            pltpu.VMEM((2, 2 * s_per, d), jnp.bfloat16),
            pltpu.SemaphoreType.DMA((2,)),
            pltpu.SemaphoreType.DMA((2,)),
            pltpu.SemaphoreType.REGULAR,
        ],
        compiler_params=pltpu.CompilerParams(collective_id=0),
    )(q, k, v)


# baseline (device time: 433654 ns/iter reference)
import functools
import math

import jax
import jax.numpy as jnp
from jax import lax
from jax.experimental import pallas as pl
from jax.experimental.pallas import tpu as pltpu

N_DEV = 32


def kernel(q, k, v):
    s_per, d = q.shape
    scale = 1.0 / math.sqrt(d)

    def body(q_ref, k_ref, v_ref, out_ref, kv_ref, send_sems, recv_sems,
             ack_buf, ack_send_sems, ack_recv_sems):
        my = lax.axis_index("i")
        left = (my - 1) % N_DEV
        right = (my + 1) % N_DEV

        barrier_sem = pltpu.get_barrier_semaphore()
        for nbr in [left, right]:
            pl.semaphore_signal(
                barrier_sem, inc=1,
                device_id=(nbr,), device_id_type=pl.DeviceIdType.MESH,
            )
        pl.semaphore_wait(barrier_sem, 2)

        kv_ref[0, pl.ds(0, s_per), :] = k_ref[:, :].astype(jnp.bfloat16)
        kv_ref[0, pl.ds(s_per, s_per), :] = v_ref[:, :].astype(jnp.bfloat16)

        q_bf = (q_ref[:, :] * scale).astype(jnp.bfloat16)

        def block_update(state, k_blk, v_blk):
            m, l, acc = state
            s = lax.dot_general(
                q_bf, k_blk, (((1,), (1,)), ((), ())),
                preferred_element_type=jnp.float32,
            )
            m_new = jnp.maximum(m, jnp.max(s, axis=1, keepdims=True))
            p = jnp.exp(s - m_new)
            alpha = jnp.exp(m - m_new)
            l_new = l * alpha + jnp.sum(p, axis=1, keepdims=True)
            acc_new = acc * alpha + lax.dot_general(
                p.astype(jnp.bfloat16), v_blk, (((1,), (0,)), ((), ())),
                preferred_element_type=jnp.float32,
            )
            return m_new, l_new, acc_new

        m0 = jnp.full((s_per, 1), -jnp.inf, dtype=jnp.float32)
        l0 = jnp.zeros((s_per, 1), dtype=jnp.float32)
        a0 = jnp.zeros((s_per, d), dtype=jnp.float32)
        state = block_update(
            (m0, l0, a0),
            k_ref[:, :].astype(jnp.bfloat16),
            v_ref[:, :].astype(jnp.bfloat16),
        )

        def make_ack(slot, dev):
            return pltpu.make_async_remote_copy(
                src_ref=ack_buf.at[slot],
                dst_ref=ack_buf.at[slot],
                send_sem=ack_send_sems.at[slot],
                recv_sem=ack_recv_sems.at[slot],
                device_id=(dev,),
                device_id_type=pl.DeviceIdType.MESH,
            )

        acks = []
        for h in range(N_DEV - 1):
            send_slot = h % 2
            recv_slot = (h + 1) % 2
            if h >= 1:
                make_ack((h - 1) % 2, right).wait_recv()
            rdma = pltpu.make_async_remote_copy(
                src_ref=kv_ref.at[send_slot],
                dst_ref=kv_ref.at[recv_slot],
                send_sem=send_sems.at[send_slot],
                recv_sem=recv_sems.at[recv_slot],
                device_id=(right,),
                device_id_type=pl.DeviceIdType.MESH,
            )
            rdma.start()
            rdma.wait()
            if h < N_DEV - 2:
                if h >= 2:
                    acks[h - 2].wait_send()
                ack = make_ack(h % 2, left)
                ack.start()
                acks.append(ack)
            state = block_update(
                state,
                kv_ref[recv_slot, pl.ds(0, s_per), :],
                kv_ref[recv_slot, pl.ds(s_per, s_per), :],
            )
        for a in acks[max(0, len(acks) - 2):]:
            a.wait_send()

        m, l, acc = state
        out_ref[:, :] = acc / l

    return pl.pallas_call(
        body,
        out_shape=jax.ShapeDtypeStruct((s_per, d), jnp.float32),
        in_specs=[
            pl.BlockSpec(memory_space=pltpu.VMEM),
            pl.BlockSpec(memory_space=pltpu.VMEM),
            pl.BlockSpec(memory_space=pltpu.VMEM),
        ],
        out_specs=pl.BlockSpec(memory_space=pltpu.VMEM),
        scratch_shapes=[
            pltpu.VMEM((2, 2 * s_per, d), jnp.bfloat16),
            pltpu.SemaphoreType.DMA((2,)),
            pltpu.SemaphoreType.DMA((2,)),
            pltpu.VMEM((2, 8, 128), jnp.float32),
            pltpu.SemaphoreType.DMA((2,)),
            pltpu.SemaphoreType.DMA((2,)),
        ],
        compiler_params=pltpu.CompilerParams(collective_id=0),
    )(q, k, v)
